# baseline (speedup 1.0000x reference)
import functools

import jax
import jax.numpy as jnp
from jax import lax
from jax.experimental import pallas as pl
from jax.experimental.pallas import tpu as pltpu
from jax.experimental.pallas import tpu_sc as plsc

N_NODES = 10000
D = 128
N_EDGES = 320000
BN_EPS = 1e-5

NC = 2
NS = 16
NW = NC * NS
CHUNK = 128
EDGES_PER_TILE = N_EDGES // NW
NITER = -(-EDGES_PER_TILE // CHUNK)
E_TILE_PAD = NITER * CHUNK
N_PAD = 10240
ROWS_PER_TILE = N_PAD // NS
NBUF = 2


def _sc_segment_sum(x, packed3):
    mesh = plsc.VectorSubcoreMesh(core_axis_name="c", subcore_axis_name="s")

    @functools.partial(
        pl.kernel,
        out_type=jax.ShapeDtypeStruct((NC, N_PAD, D), jnp.float32),
        mesh=mesh,
        scratch_types=[
            pltpu.VMEM((NBUF, CHUNK), jnp.int32),
            pltpu.VMEM((NBUF, CHUNK), jnp.int32),
            pltpu.VMEM((NBUF, CHUNK), jnp.int32),
            pltpu.VMEM((NBUF, CHUNK, D), jnp.float32),
            pltpu.VMEM_SHARED((N_PAD, D), jnp.float32),
            pltpu.SemaphoreType.DMA((NBUF,)),
            pltpu.SemaphoreType.DMA((NBUF,)),
            pltpu.SemaphoreType.DMA((NBUF,)),
        ],
    )
    def seg_sum(x_hbm, pk_hbm, out_hbm,
                pk_v, su_v, du_v, rows_v, agg_sh, gsem, ssem, isem):
        c = lax.axis_index("c")
        s = lax.axis_index("s")
        wid = s * NC + c

        zv = jnp.zeros((16,), jnp.float32)

        def zrow(r, carry):
            for t in range(D // 16):
                rows_v[0, r, pl.ds(16 * t, 16)] = zv
            return carry

        lax.fori_loop(0, CHUNK, zrow, 0)
        for t in range(ROWS_PER_TILE // CHUNK):
            pltpu.sync_copy(
                rows_v.at[0],
                agg_sh.at[pl.ds(s * ROWS_PER_TILE + t * CHUNK, CHUNK)])
        plsc.subcore_barrier()

        def ifetch(j, k):
            pltpu.async_copy(pk_hbm.at[wid].at[j], pk_v.at[k], isem.at[k])

        def ifetch_wait(j, k):
            pltpu.make_async_copy(pk_hbm.at[wid].at[j], pk_v.at[k],
                                  isem.at[k]).wait()

        def unpack_and_gather(k):
            for t in range(CHUNK // 16):
                w = pk_v[k, pl.ds(t * 16, 16)]
                su_v[k, pl.ds(t * 16, 16)] = lax.bitwise_and(w, 0xFFFF)
                du_v[k, pl.ds(t * 16, 16)] = lax.shift_right_logical(w, 16)
            pltpu.async_copy(x_hbm.at[su_v.at[k]], rows_v.at[k], gsem.at[k])

        def gather_wait(k):
            pltpu.make_async_copy(x_hbm.at[su_v.at[k]], rows_v.at[k],
                                  gsem.at[k]).wait()

        def scatter(k):
            pltpu.async_copy(rows_v.at[k], agg_sh.at[du_v.at[k]], ssem.at[k],
                             add=True)

        def scatter_wait(k):
            pltpu.make_async_copy(rows_v.at[k], agg_sh.at[du_v.at[k]],
                                  ssem.at[k]).wait()

        ifetch(0, 0)
        ifetch_wait(0, 0)
        unpack_and_gather(0)
        ifetch(1, 1)

        def step(j, k):
            @pl.when(j >= 1)
            def _():
                scatter_wait(1 - k)

            @pl.when(j + 1 < NITER)
            def _():
                ifetch_wait(j + 1, 1 - k)
                unpack_and_gather(1 - k)

            @pl.when(j + 2 < NITER)
            def _():
                ifetch(j + 2, k)

            gather_wait(k)
            scatter(k)

        def body(i, carry):
            for k in range(NBUF):
                j = NBUF * i + k

                @pl.when(j < NITER)
                def _(j=j, k=k):
                    step(j, k)
            return carry

        lax.fori_loop(0, (NITER + NBUF - 1) // NBUF, body, 0)
        scatter_wait((NITER - 1) % NBUF)
        plsc.subcore_barrier()

        r0 = s * ROWS_PER_TILE
        pltpu.sync_copy(agg_sh.at[pl.ds(r0, ROWS_PER_TILE)],
                        out_hbm.at[c].at[pl.ds(r0, ROWS_PER_TILE)])

    return seg_sum(x, packed3)


def _tc_mlp_bn(x, partials, W1, b1, W2, b2, gamma, beta):
    def body(x_ref, p_ref, w1_ref, b1_ref, w2_ref, b2_ref, g_ref, bt_ref, o_ref):
        h = x_ref[...] + p_ref[0, :N_NODES] + p_ref[1, :N_NODES]
        h = jnp.dot(h, w1_ref[...], preferred_element_type=jnp.float32) + b1_ref[...]
        h = jnp.maximum(h, 0.0)
        h = jnp.dot(h, w2_ref[...], preferred_element_type=jnp.float32) + b2_ref[...]
        h = jnp.maximum(h, 0.0)
        mean = jnp.sum(h, axis=0, keepdims=True) * (1.0 / N_NODES)
        d0 = h - mean
        var = jnp.sum(d0 * d0, axis=0, keepdims=True) * (1.0 / N_NODES)
        inv = lax.rsqrt(var + BN_EPS)
        o_ref[...] = g_ref[...] * d0 * inv + bt_ref[...]

    return pl.pallas_call(
        body,
        out_shape=jax.ShapeDtypeStruct((N_NODES, D), jnp.float32),
    )(x, partials, W1, b1, W2, b2, gamma, beta)


@jax.jit
def kernel(x, edge_index, edge_attr, W1, b1, W2, b2, gamma, beta):
    src = edge_index[0].astype(jnp.int32)
    dst = edge_index[1].astype(jnp.int32)
    packed = (src | (dst << 16)).reshape(NW, EDGES_PER_TILE)
    pad_word = N_NODES << 16
    packed3 = jnp.pad(packed, ((0, 0), (0, E_TILE_PAD - EDGES_PER_TILE)),
                      constant_values=pad_word).reshape(NW, NITER, CHUNK)
    partials = _sc_segment_sum(x, packed3)
    return _tc_mlp_bn(x, partials,
                      W1, b1.reshape(1, D), W2, b2.reshape(1, D),
                      gamma.reshape(1, D), beta.reshape(1, D))


# speedup vs baseline: 1.7560x; 1.7560x over previous
import functools

import jax
import jax.numpy as jnp
from jax import lax
from jax.experimental import pallas as pl
from jax.experimental.pallas import tpu as pltpu
from jax.experimental.pallas import tpu_sc as plsc

N_NODES = 10000
D = 128
N_EDGES = 320000
BN_EPS = 1e-5

NC = 2
NS = 16
NW = NC * NS
CHUNK = 80
EDGES_PER_TILE = N_EDGES // NW
NITER = -(-EDGES_PER_TILE // CHUNK)
E_TILE_PAD = NITER * CHUNK
N_PAD = 10240
ROWS_PER_TILE = N_PAD // NS
NBUF = 2


def _sc_segment_sum(x, packed3):
    mesh = plsc.VectorSubcoreMesh(core_axis_name="c", subcore_axis_name="s")

    @functools.partial(
        pl.kernel,
        out_type=jax.ShapeDtypeStruct((NC, N_PAD, D), jnp.float32),
        mesh=mesh,
        scratch_types=[
            pltpu.VMEM((NBUF, CHUNK), jnp.int32),
            pltpu.VMEM((NBUF, CHUNK), jnp.int32),
            pltpu.VMEM((NBUF, CHUNK), jnp.int32),
            pltpu.VMEM((NBUF, CHUNK, D), jnp.float32),
            pltpu.VMEM_SHARED((N_PAD, D), jnp.float32),
            pltpu.SemaphoreType.DMA((NBUF,)),
            pltpu.SemaphoreType.DMA((NBUF,)),
            pltpu.SemaphoreType.DMA((NBUF,)),
        ],
    )
    def seg_sum(x_hbm, pk_hbm, out_hbm,
                pk_v, su_v, du_v, rows_v, agg_sh, gsem, ssem, isem):
        c = lax.axis_index("c")
        s = lax.axis_index("s")
        wid = s * NC + c

        zv = jnp.zeros((16,), jnp.float32)

        def zrow(r, carry):
            for t in range(D // 16):
                rows_v[0, r, pl.ds(16 * t, 16)] = zv
            return carry

        lax.fori_loop(0, CHUNK, zrow, 0)
        for t in range(ROWS_PER_TILE // CHUNK):
            pltpu.sync_copy(
                rows_v.at[0],
                agg_sh.at[pl.ds(s * ROWS_PER_TILE + t * CHUNK, CHUNK)])
        plsc.subcore_barrier()

        def ifetch(j, k):
            pltpu.async_copy(pk_hbm.at[wid].at[j], pk_v.at[k], isem.at[k])

        def ifetch_wait(j, k):
            pltpu.make_async_copy(pk_hbm.at[wid].at[j], pk_v.at[k],
                                  isem.at[k]).wait()

        def unpack_and_gather(k):
            for t in range(CHUNK // 16):
                w = pk_v[k, pl.ds(t * 16, 16)]
                su_v[k, pl.ds(t * 16, 16)] = lax.bitwise_and(w, 0xFFFF)
                du_v[k, pl.ds(t * 16, 16)] = lax.shift_right_logical(w, 16)
            pltpu.async_copy(x_hbm.at[su_v.at[k]], rows_v.at[k], gsem.at[k])

        def gather_wait(k):
            pltpu.make_async_copy(x_hbm.at[su_v.at[k]], rows_v.at[k],
                                  gsem.at[k]).wait()

        def scatter(k):
            pltpu.async_copy(rows_v.at[k], agg_sh.at[du_v.at[k]], ssem.at[k],
                             add=True)

        def scatter_wait(k):
            pltpu.make_async_copy(rows_v.at[k], agg_sh.at[du_v.at[k]],
                                  ssem.at[k]).wait()

        ifetch(0, 0)
        ifetch_wait(0, 0)
        unpack_and_gather(0)
        ifetch(1, 1)

        def step(j, k):
            @pl.when(j >= 1)
            def _():
                scatter_wait(1 - k)

            @pl.when(j + 1 < NITER)
            def _():
                ifetch_wait(j + 1, 1 - k)
                unpack_and_gather(1 - k)

            @pl.when(j + 2 < NITER)
            def _():
                ifetch(j + 2, k)

            gather_wait(k)
            scatter(k)

        def body(i, carry):
            for k in range(NBUF):
                j = NBUF * i + k

                @pl.when(j < NITER)
                def _(j=j, k=k):
                    step(j, k)
            return carry

        lax.fori_loop(0, (NITER + NBUF - 1) // NBUF, body, 0)
        scatter_wait((NITER - 1) % NBUF)
        plsc.subcore_barrier()

        r0 = s * ROWS_PER_TILE
        pltpu.sync_copy(agg_sh.at[pl.ds(r0, ROWS_PER_TILE)],
                        out_hbm.at[c].at[pl.ds(r0, ROWS_PER_TILE)])

    return seg_sum(x, packed3)


def _tc_mlp_bn(x, partials, W1, b1, W2, b2, gamma, beta):
    def body(x_ref, p_ref, w1_ref, b1_ref, w2_ref, b2_ref, g_ref, bt_ref, o_ref):
        h = x_ref[...] + p_ref[0, :N_NODES] + p_ref[1, :N_NODES]
        h = jnp.dot(h, w1_ref[...], preferred_element_type=jnp.float32) + b1_ref[...]
        h = jnp.maximum(h, 0.0)
        h = jnp.dot(h, w2_ref[...], preferred_element_type=jnp.float32) + b2_ref[...]
        h = jnp.maximum(h, 0.0)
        mean = jnp.sum(h, axis=0, keepdims=True) * (1.0 / N_NODES)
        d0 = h - mean
        var = jnp.sum(d0 * d0, axis=0, keepdims=True) * (1.0 / N_NODES)
        inv = lax.rsqrt(var + BN_EPS)
        o_ref[...] = g_ref[...] * d0 * inv + bt_ref[...]

    return pl.pallas_call(
        body,
        out_shape=jax.ShapeDtypeStruct((N_NODES, D), jnp.float32),
    )(x, partials, W1, b1, W2, b2, gamma, beta)


@jax.jit
def kernel(x, edge_index, edge_attr, W1, b1, W2, b2, gamma, beta):
    src = edge_index[0].astype(jnp.int32)
    dst = edge_index[1].astype(jnp.int32)
    packed = (src | (dst << 16)).reshape(NW, EDGES_PER_TILE)
    pad_word = N_NODES << 16
    packed3 = jnp.pad(packed, ((0, 0), (0, E_TILE_PAD - EDGES_PER_TILE)),
                      constant_values=pad_word).reshape(NW, NITER, CHUNK)
    partials = _sc_segment_sum(x, packed3)
    return _tc_mlp_bn(x, partials,
                      W1, b1.reshape(1, D), W2, b2.reshape(1, D),
                      gamma.reshape(1, D), beta.reshape(1, D))


# speedup vs baseline: 2.0402x; 1.1618x over previous
import functools

import jax
import jax.numpy as jnp
from jax import lax
from jax.experimental import pallas as pl
from jax.experimental.pallas import tpu as pltpu
from jax.experimental.pallas import tpu_sc as plsc

N_NODES = 10000
D = 128
N_EDGES = 320000
BN_EPS = 1e-5

NC = 2
NS = 16
NW = NC * NS
CHUNK = 80
EDGES_PER_TILE = N_EDGES // NW
NITER = -(-EDGES_PER_TILE // CHUNK)
E_TILE_PAD = NITER * CHUNK
N_PAD = 10240
ROWS_PER_TILE = N_PAD // NS
NBUF = 3


def _sc_segment_sum(x, packed3):
    mesh = plsc.VectorSubcoreMesh(core_axis_name="c", subcore_axis_name="s")

    @functools.partial(
        pl.kernel,
        out_type=jax.ShapeDtypeStruct((NC, N_PAD, D), jnp.float32),
        mesh=mesh,
        scratch_types=[
            pltpu.VMEM((NBUF, CHUNK), jnp.int32),
            pltpu.VMEM((NBUF, CHUNK), jnp.int32),
            pltpu.VMEM((NBUF, CHUNK), jnp.int32),
            pltpu.VMEM((NBUF, CHUNK, D), jnp.float32),
            pltpu.VMEM_SHARED((N_PAD, D), jnp.float32),
            pltpu.SemaphoreType.DMA((NBUF,)),
            pltpu.SemaphoreType.DMA((NBUF,)),
            pltpu.SemaphoreType.DMA((NBUF,)),
        ],
    )
    def seg_sum(x_hbm, pk_hbm, out_hbm,
                pk_v, su_v, du_v, rows_v, agg_sh, gsem, ssem, isem):
        c = lax.axis_index("c")
        s = lax.axis_index("s")
        wid = s * NC + c

        zv = jnp.zeros((16,), jnp.float32)

        def zrow(r, carry):
            for t in range(D // 16):
                rows_v[0, r, pl.ds(16 * t, 16)] = zv
            return carry

        lax.fori_loop(0, CHUNK, zrow, 0)
        for t in range(ROWS_PER_TILE // CHUNK):
            pltpu.sync_copy(
                rows_v.at[0],
                agg_sh.at[pl.ds(s * ROWS_PER_TILE + t * CHUNK, CHUNK)])
        plsc.subcore_barrier()

        def ifetch(j, k):
            pltpu.async_copy(pk_hbm.at[wid].at[j], pk_v.at[k], isem.at[k])

        def ifetch_wait(j, k):
            pltpu.make_async_copy(pk_hbm.at[wid].at[j], pk_v.at[k],
                                  isem.at[k]).wait()

        def unpack_and_gather(k):
            for t in range(CHUNK // 16):
                w = pk_v[k, pl.ds(t * 16, 16)]
                su_v[k, pl.ds(t * 16, 16)] = lax.bitwise_and(w, 0xFFFF)
                du_v[k, pl.ds(t * 16, 16)] = lax.shift_right_logical(w, 16)
            pltpu.async_copy(x_hbm.at[su_v.at[k]], rows_v.at[k], gsem.at[k])

        def gather_wait(k):
            pltpu.make_async_copy(x_hbm.at[su_v.at[k]], rows_v.at[k],
                                  gsem.at[k]).wait()

        def scatter(k):
            pltpu.async_copy(rows_v.at[k], agg_sh.at[du_v.at[k]], ssem.at[k],
                             add=True)

        def scatter_wait(k):
            pltpu.make_async_copy(rows_v.at[k], agg_sh.at[du_v.at[k]],
                                  ssem.at[k]).wait()

        ifetch(0, 0)
        ifetch_wait(0, 0)
        unpack_and_gather(0)
        ifetch(1, 1)
        ifetch_wait(1, 1)
        unpack_and_gather(1)
        ifetch(2, 2)

        def step(j, k):
            @pl.when(j >= 1)
            def _():
                scatter_wait((k + 2) % NBUF)

            @pl.when(j + 2 < NITER)
            def _():
                ifetch_wait(j + 2, (k + 2) % NBUF)
                unpack_and_gather((k + 2) % NBUF)

            @pl.when(j + 3 < NITER)
            def _():
                ifetch(j + 3, k)

            gather_wait(k)
            scatter(k)

        def body(i, carry):
            for k in range(NBUF):
                j = NBUF * i + k

                @pl.when(j < NITER)
                def _(j=j, k=k):
                    step(j, k)
            return carry

        lax.fori_loop(0, (NITER + NBUF - 1) // NBUF, body, 0)
        scatter_wait((NITER - 1) % NBUF)
        plsc.subcore_barrier()

        r0 = s * ROWS_PER_TILE
        pltpu.sync_copy(agg_sh.at[pl.ds(r0, ROWS_PER_TILE)],
                        out_hbm.at[c].at[pl.ds(r0, ROWS_PER_TILE)])

    return seg_sum(x, packed3)


def _tc_mlp_bn(x, partials, W1, b1, W2, b2, gamma, beta):
    def body(x_ref, p_ref, w1_ref, b1_ref, w2_ref, b2_ref, g_ref, bt_ref, o_ref):
        h = x_ref[...] + p_ref[0, :N_NODES] + p_ref[1, :N_NODES]
        h = jnp.dot(h, w1_ref[...], preferred_element_type=jnp.float32) + b1_ref[...]
        h = jnp.maximum(h, 0.0)
        h = jnp.dot(h, w2_ref[...], preferred_element_type=jnp.float32) + b2_ref[...]
        h = jnp.maximum(h, 0.0)
        mean = jnp.sum(h, axis=0, keepdims=True) * (1.0 / N_NODES)
        d0 = h - mean
        var = jnp.sum(d0 * d0, axis=0, keepdims=True) * (1.0 / N_NODES)
        inv = lax.rsqrt(var + BN_EPS)
        o_ref[...] = g_ref[...] * d0 * inv + bt_ref[...]

    return pl.pallas_call(
        body,
        out_shape=jax.ShapeDtypeStruct((N_NODES, D), jnp.float32),
    )(x, partials, W1, b1, W2, b2, gamma, beta)


@jax.jit
def kernel(x, edge_index, edge_attr, W1, b1, W2, b2, gamma, beta):
    src = edge_index[0].astype(jnp.int32)
    dst = edge_index[1].astype(jnp.int32)
    packed = (src | (dst << 16)).reshape(NW, EDGES_PER_TILE)
    pad_word = N_NODES << 16
    packed3 = jnp.pad(packed, ((0, 0), (0, E_TILE_PAD - EDGES_PER_TILE)),
                      constant_values=pad_word).reshape(NW, NITER, CHUNK)
    partials = _sc_segment_sum(x, packed3)
    return _tc_mlp_bn(x, partials,
                      W1, b1.reshape(1, D), W2, b2.reshape(1, D),
                      gamma.reshape(1, D), beta.reshape(1, D))


# speedup vs baseline: 2.1741x; 1.0657x over previous
import functools

import jax
import jax.numpy as jnp
from jax import lax
from jax.experimental import pallas as pl
from jax.experimental.pallas import tpu as pltpu
from jax.experimental.pallas import tpu_sc as plsc

N_NODES = 10000
D = 128
N_EDGES = 320000
BN_EPS = 1e-5

NC = 2
NS = 16
NW = NC * NS
CHUNK = 80
EDGES_PER_TILE = N_EDGES // NW
NITER = -(-EDGES_PER_TILE // CHUNK)
E_TILE_PAD = NITER * CHUNK
N_PAD = 10240
ROWS_PER_TILE = N_PAD // NS
NBUF = 3


def _sc_segment_sum(x, ei4):
    mesh = plsc.VectorSubcoreMesh(core_axis_name="c", subcore_axis_name="s")

    @functools.partial(
        pl.kernel,
        out_type=jax.ShapeDtypeStruct((NC, N_PAD, D), jnp.float32),
        mesh=mesh,
        scratch_types=[
            pltpu.VMEM((NBUF, CHUNK), jnp.int32),
            pltpu.VMEM((NBUF, CHUNK), jnp.int32),
            pltpu.VMEM((NBUF, CHUNK, D), jnp.float32),
            pltpu.VMEM_SHARED((N_PAD, D), jnp.float32),
            pltpu.SemaphoreType.DMA((NBUF,)),
            pltpu.SemaphoreType.DMA((NBUF,)),
            pltpu.SemaphoreType.DMA((NBUF,)),
            pltpu.SemaphoreType.DMA((NBUF,)),
        ],
    )
    def seg_sum(x_hbm, ei_hbm, out_hbm,
                su_v, du_v, rows_v, agg_sh, gsem, ssem, isem, dsem):
        c = lax.axis_index("c")
        s = lax.axis_index("s")
        wid = s * NC + c

        zv = jnp.zeros((16,), jnp.float32)

        def zrow(r, carry):
            for t in range(D // 16):
                rows_v[0, r, pl.ds(16 * t, 16)] = zv
            return carry

        lax.fori_loop(0, CHUNK, zrow, 0)
        for t in range(ROWS_PER_TILE // CHUNK):
            pltpu.sync_copy(
                rows_v.at[0],
                agg_sh.at[pl.ds(s * ROWS_PER_TILE + t * CHUNK, CHUNK)])
        plsc.subcore_barrier()

        def ifetch(j, k):
            pltpu.async_copy(ei_hbm.at[0].at[wid].at[j], su_v.at[k], isem.at[k])
            pltpu.async_copy(ei_hbm.at[1].at[wid].at[j], du_v.at[k], dsem.at[k])

        def ifetch_wait(j, k):
            pltpu.make_async_copy(ei_hbm.at[0].at[wid].at[j], su_v.at[k],
                                  isem.at[k]).wait()
            pltpu.make_async_copy(ei_hbm.at[1].at[wid].at[j], du_v.at[k],
                                  dsem.at[k]).wait()

        def unpack_and_gather(k):
            pltpu.async_copy(x_hbm.at[su_v.at[k]], rows_v.at[k], gsem.at[k])

        def gather_wait(k):
            pltpu.make_async_copy(x_hbm.at[su_v.at[k]], rows_v.at[k],
                                  gsem.at[k]).wait()

        def scatter(k):
            pltpu.async_copy(rows_v.at[k], agg_sh.at[du_v.at[k]], ssem.at[k],
                             add=True)

        def scatter_wait(k):
            pltpu.make_async_copy(rows_v.at[k], agg_sh.at[du_v.at[k]],
                                  ssem.at[k]).wait()

        ifetch(0, 0)
        ifetch_wait(0, 0)
        unpack_and_gather(0)
        ifetch(1, 1)
        ifetch_wait(1, 1)
        unpack_and_gather(1)
        ifetch(2, 2)

        def step(j, k):
            @pl.when(j >= 1)
            def _():
                scatter_wait((k + 2) % NBUF)

            @pl.when(j + 2 < NITER)
            def _():
                ifetch_wait(j + 2, (k + 2) % NBUF)
                unpack_and_gather((k + 2) % NBUF)

            @pl.when(j + 3 < NITER)
            def _():
                ifetch(j + 3, k)

            gather_wait(k)
            scatter(k)

        def body(i, carry):
            for k in range(NBUF):
                j = NBUF * i + k

                @pl.when(j < NITER)
                def _(j=j, k=k):
                    step(j, k)
            return carry

        lax.fori_loop(0, (NITER + NBUF - 1) // NBUF, body, 0)
        scatter_wait((NITER - 1) % NBUF)
        plsc.subcore_barrier()

        r0 = s * ROWS_PER_TILE
        pltpu.sync_copy(agg_sh.at[pl.ds(r0, ROWS_PER_TILE)],
                        out_hbm.at[c].at[pl.ds(r0, ROWS_PER_TILE)])

    return seg_sum(x, ei4)


def _tc_mlp_bn(x, partials, W1, b1, W2, b2, gamma, beta):
    def body(x_ref, p_ref, w1_ref, b1_ref, w2_ref, b2_ref, g_ref, bt_ref, o_ref):
        h = x_ref[...] + p_ref[0, :N_NODES] + p_ref[1, :N_NODES]
        h = jnp.dot(h, w1_ref[...], preferred_element_type=jnp.float32) + b1_ref[...]
        h = jnp.maximum(h, 0.0)
        h = jnp.dot(h, w2_ref[...], preferred_element_type=jnp.float32) + b2_ref[...]
        h = jnp.maximum(h, 0.0)
        mean = jnp.sum(h, axis=0, keepdims=True) * (1.0 / N_NODES)
        d0 = h - mean
        var = jnp.sum(d0 * d0, axis=0, keepdims=True) * (1.0 / N_NODES)
        inv = lax.rsqrt(var + BN_EPS)
        o_ref[...] = g_ref[...] * d0 * inv + bt_ref[...]

    return pl.pallas_call(
        body,
        out_shape=jax.ShapeDtypeStruct((N_NODES, D), jnp.float32),
    )(x, partials, W1, b1, W2, b2, gamma, beta)


@jax.jit
def kernel(x, edge_index, edge_attr, W1, b1, W2, b2, gamma, beta):
    ei4 = edge_index.astype(jnp.int32).reshape(2, NW, NITER, CHUNK)
    partials = _sc_segment_sum(x, ei4)
    return _tc_mlp_bn(x, partials,
                      W1, b1.reshape(1, D), W2, b2.reshape(1, D),
                      gamma.reshape(1, D), beta.reshape(1, D))
